# Initial kernel scaffold; baseline (speedup 1.0000x reference)
import functools

import jax
import jax.numpy as jnp
from jax import lax
from jax.experimental import pallas as pl
from jax.experimental.pallas import tpu as pltpu
from jax.experimental.pallas import tpu_sc as plsc

NN = 10000
EE = 320000
DD = 128
HH = 4
CC = 32
HC = HH * CC
NEG = 0.2

NCORE = 2
NSUB = 16
NTILE = NCORE * NSUB

CHUNK = 128
ETOT = EE + NN
NCHUNK = -(-ETOT // (NTILE * CHUNK))
EPT = NCHUNK * CHUNK
EPAD = NTILE * EPT

ACC_ROWS = 10240
RPT = ACC_ROWS // NSUB
DEN_W = 16


def _mm_body(x_ref, ws_ref, wd_ref, xs_ref, xd_ref):
    x = x_ref[...]
    xs_ref[...] = jnp.dot(x, ws_ref[...], preferred_element_type=jnp.float32)
    xd_ref[...] = jnp.dot(x, wd_ref[...], preferred_element_type=jnp.float32)


def _project(x, w_src, w_dst):
    rows = 1000
    grid = NN // rows
    return pl.pallas_call(
        _mm_body,
        grid=(grid,),
        in_specs=[
            pl.BlockSpec((rows, DD), lambda i: (i, 0)),
            pl.BlockSpec((DD, HC), lambda i: (0, 0)),
            pl.BlockSpec((DD, HC), lambda i: (0, 0)),
        ],
        out_specs=[
            pl.BlockSpec((rows, HC), lambda i: (i, 0)),
            pl.BlockSpec((rows, HC), lambda i: (i, 0)),
        ],
        out_shape=[jax.ShapeDtypeStruct((NN, HC), jnp.float32)] * 2,
    )(x, w_src, w_dst)


def _sc_edges_body(xs_hbm, xd_hbm, src_hbm, dst_hbm, att_hbm,
                   num_out, den_out,
                   src_v, dst_v, xs_b, xd_b, msg_b, den_b, att_v,
                   num_acc, den_acc, sem0, sem1):
    cid = lax.axis_index("c")
    sid = lax.axis_index("s")
    zero16 = jnp.zeros((16,), jnp.float32)

    def _zrow(r, c):
        for j in range(HC // 16):
            msg_b[r, pl.ds(16 * j, 16)] = zero16
        den_b[r, :] = zero16
        return c
    lax.fori_loop(0, CHUNK, _zrow, 0)

    row0 = sid * RPT
    for j in range(RPT // CHUNK):
        pltpu.sync_copy(msg_b, num_acc.at[pl.ds(row0 + j * CHUNK, CHUNK)])
        pltpu.sync_copy(den_b, den_acc.at[pl.ds(row0 + j * CHUNK, CHUNK)])
    plsc.subcore_barrier()

    pltpu.sync_copy(att_hbm, att_v)
    att_regs = [att_v[pl.ds(16 * j, 16)] for j in range(HC // 16)]
    lane = lax.broadcasted_iota(jnp.int32, (16,), 0)

    tile = cid * NSUB + sid
    base = tile * EPT

    def _chunk(cn, c):
        off = pl.multiple_of(base + cn * CHUNK, CHUNK)
        pltpu.sync_copy(src_hbm.at[pl.ds(off, CHUNK)], src_v)
        pltpu.sync_copy(dst_hbm.at[pl.ds(off, CHUNK)], dst_v)
        cp0 = pltpu.async_copy(xs_hbm.at[src_v], xs_b, sem0)
        cp1 = pltpu.async_copy(xd_hbm.at[dst_v], xd_b, sem1)
        cp0.wait()
        cp1.wait()

        def _edge(e, cc2):
            rs = [xs_b[e, pl.ds(16 * j, 16)] for j in range(HC // 16)]
            rd = [xd_b[e, pl.ds(16 * j, 16)] for j in range(HC // 16)]
            ps = []
            for h in range(HH):
                acc = None
                for k in (2 * h, 2 * h + 1):
                    s = rs[k] + rd[k]
                    l = jnp.where(s >= 0.0, s, s * NEG) * att_regs[k]
                    acc = l if acc is None else acc + l
                r = jnp.sum(acc)
                ps.append(jnp.exp(jnp.full((16,), r, jnp.float32)))
            for j in range(HC // 16):
                msg_b[e, pl.ds(16 * j, 16)] = rs[j] * ps[j // 2]
            dv = jnp.where(lane == 0, ps[0], 0.0)
            dv = jnp.where(lane == 1, ps[1], dv)
            dv = jnp.where(lane == 2, ps[2], dv)
            dv = jnp.where(lane == 3, ps[3], dv)
            den_b[e, :] = dv
            return cc2
        lax.fori_loop(0, CHUNK, _edge, 0)

        pltpu.sync_copy(msg_b, num_acc.at[dst_v], add=True)
        pltpu.sync_copy(den_b, den_acc.at[dst_v], add=True)
        return c
    lax.fori_loop(0, NCHUNK, _chunk, 0)

    plsc.subcore_barrier()
    pltpu.sync_copy(num_acc.at[pl.ds(row0, RPT)],
                    num_out.at[cid, pl.ds(row0, RPT)])
    pltpu.sync_copy(den_acc.at[pl.ds(row0, RPT)],
                    den_out.at[cid, pl.ds(row0, RPT)])


def _sc_edges(xs, xd, src, dst, att_flat):
    mesh = plsc.VectorSubcoreMesh(core_axis_name="c", subcore_axis_name="s")
    return pl.kernel(
        _sc_edges_body,
        out_type=[
            jax.ShapeDtypeStruct((NCORE, ACC_ROWS, HC), jnp.float32),
            jax.ShapeDtypeStruct((NCORE, ACC_ROWS, DEN_W), jnp.float32),
        ],
        mesh=mesh,
        scratch_types=[
            pltpu.VMEM((CHUNK,), jnp.int32),
            pltpu.VMEM((CHUNK,), jnp.int32),
            pltpu.VMEM((CHUNK, HC), jnp.float32),
            pltpu.VMEM((CHUNK, HC), jnp.float32),
            pltpu.VMEM((CHUNK, HC), jnp.float32),
            pltpu.VMEM((CHUNK, DEN_W), jnp.float32),
            pltpu.VMEM((HC,), jnp.float32),
            pltpu.VMEM_SHARED((ACC_ROWS, HC), jnp.float32),
            pltpu.VMEM_SHARED((ACC_ROWS, DEN_W), jnp.float32),
            pltpu.SemaphoreType.DMA,
            pltpu.SemaphoreType.DMA,
        ],
    )(xs, xd, src, dst, att_flat)


def _combine_body(num_ref, den_ref, bias_ref, out_ref):
    num = num_ref[0] + num_ref[1]
    den = den_ref[0] + den_ref[1]
    row = lax.broadcasted_iota(jnp.int32, (DEN_W, HC), 0)
    col = lax.broadcasted_iota(jnp.int32, (DEN_W, HC), 1)
    sel = jnp.where(row == col // CC, 1.0, 0.0)
    den_b = jnp.dot(den, sel, preferred_element_type=jnp.float32)
    out_ref[...] = num / den_b + bias_ref[...]


def _combine(num, den, bias2d):
    rows = 500
    grid = NN // rows
    return pl.pallas_call(
        _combine_body,
        grid=(grid,),
        in_specs=[
            pl.BlockSpec((NCORE, rows, HC), lambda i: (0, i, 0)),
            pl.BlockSpec((NCORE, rows, DEN_W), lambda i: (0, i, 0)),
            pl.BlockSpec((1, HC), lambda i: (0, 0)),
        ],
        out_specs=pl.BlockSpec((rows, HC), lambda i: (i, 0)),
        out_shape=jax.ShapeDtypeStruct((NN, HC), jnp.float32),
    )(num, den, bias2d)


@jax.jit
def kernel(x, edge_index, W_src, W_dst, att, bias):
    xs, xd = _project(x, W_src, W_dst)
    loops = jnp.arange(NN, dtype=jnp.int32)
    pad = EPAD - ETOT
    src = jnp.concatenate(
        [edge_index[0].astype(jnp.int32), loops,
         jnp.zeros((pad,), jnp.int32)])
    dst = jnp.concatenate(
        [edge_index[1].astype(jnp.int32), loops,
         jnp.full((pad,), NN, jnp.int32)])
    att_flat = att.reshape(HC)
    num, den = _sc_edges(xs, xd, src, dst, att_flat)
    out = _combine(num, den, bias.reshape(1, HC))
    return out


# speedup vs baseline: 36.3520x; 36.3520x over previous
import jax
import jax.numpy as jnp
from jax import lax
from jax.experimental import pallas as pl
from jax.experimental.pallas import tpu as pltpu
from jax.experimental.pallas import tpu_sc as plsc

NN = 10000
EE = 320000
DD = 128
HH = 4
CC = 32
HC = HH * CC
NEG = 0.2

NCORE = 2
NSUB = 16
NTILE = NCORE * NSUB

CHUNK = 64
ETOT = EE + NN
NCHUNK = -(-ETOT // (NTILE * CHUNK))
EPT = NCHUNK * CHUNK
EPAD = NTILE * EPT

NUM_ROWS = 10240
DEN_ROWS = 1280
ACC_T = NUM_ROWS + DEN_ROWS
NRPT = NUM_ROWS // NSUB
DRPT = DEN_ROWS // NSUB


def _mm_body(x_ref, ws_ref, wd_ref, xs_ref, xd_ref):
    x = x_ref[...]
    xs_ref[...] = jnp.dot(x, ws_ref[...], preferred_element_type=jnp.float32)
    xd_ref[...] = jnp.dot(x, wd_ref[...], preferred_element_type=jnp.float32)


def _project(x, w_src, w_dst):
    rows = 1000
    grid = NN // rows
    return pl.pallas_call(
        _mm_body,
        grid=(grid,),
        in_specs=[
            pl.BlockSpec((rows, DD), lambda i: (i, 0)),
            pl.BlockSpec((DD, HC), lambda i: (0, 0)),
            pl.BlockSpec((DD, HC), lambda i: (0, 0)),
        ],
        out_specs=[
            pl.BlockSpec((rows, HC), lambda i: (i, 0)),
            pl.BlockSpec((rows, HC), lambda i: (i, 0)),
        ],
        out_shape=[jax.ShapeDtypeStruct((NN, HC), jnp.float32)] * 2,
    )(x, w_src, w_dst)


def _sc_edges_body(xs_hbm, xd_hbm, src_hbm, dst_hbm, att_hbm,
                   num_out, den_out,
                   src_v, dst_v, didx_v, xs_b, xd_b, msg_b, den_b, att_v,
                   acc, sem0, sem1):
    cid = lax.axis_index("c")
    sid = lax.axis_index("s")
    zero16 = jnp.zeros((16,), jnp.float32)

    def _zrow(r, c):
        for j in range(HC // 16):
            msg_b[r, pl.ds(16 * j, 16)] = zero16
        return c
    lax.fori_loop(0, CHUNK, _zrow, 0)

    row0 = sid * NRPT
    for j in range(NUM_ROWS // NSUB // CHUNK):
        pltpu.sync_copy(msg_b, acc.at[pl.ds(row0 + j * CHUNK, CHUNK)])
    d0 = NUM_ROWS + sid * DRPT
    pltpu.sync_copy(msg_b, acc.at[pl.ds(d0, CHUNK)])
    pltpu.sync_copy(msg_b, acc.at[pl.ds(d0 + DRPT - CHUNK, CHUNK)])
    plsc.subcore_barrier()

    pltpu.sync_copy(att_hbm, att_v)
    att_regs = [att_v[pl.ds(16 * j, 16)] for j in range(HC // 16)]
    lane = lax.broadcasted_iota(jnp.int32, (16,), 0)

    tile = cid * NSUB + sid
    base = tile * EPT

    def _chunk(cn, c):
        off = pl.multiple_of(base + cn * CHUNK, CHUNK)
        pltpu.sync_copy(src_hbm.at[pl.ds(off, CHUNK)], src_v)
        pltpu.sync_copy(dst_hbm.at[pl.ds(off, CHUNK)], dst_v)
        cp0 = pltpu.async_copy(xs_hbm.at[src_v], xs_b, sem0)
        cp1 = pltpu.async_copy(xd_hbm.at[dst_v], xd_b, sem1)
        for j in range(CHUNK // 16):
            dv = dst_v[pl.ds(16 * j, 16)]
            didx_v[pl.ds(16 * j, 16)] = NUM_ROWS + (dv >> 3)
        cp0.wait()
        cp1.wait()

        def _edge(e, cc2):
            rs = [xs_b[e, pl.ds(16 * j, 16)] for j in range(HC // 16)]
            rd = [xd_b[e, pl.ds(16 * j, 16)] for j in range(HC // 16)]
            ps = []
            for h in range(HH):
                acc_h = None
                for k in (2 * h, 2 * h + 1):
                    s = rs[k] + rd[k]
                    l = jnp.where(s >= 0.0, s, s * NEG) * att_regs[k]
                    acc_h = l if acc_h is None else acc_h + l
                for sh in (1, 2, 4, 8):
                    acc_h = acc_h + acc_h.at[lane ^ sh].get(
                        mode="promise_in_bounds")
                ps.append(jnp.exp(acc_h))
            for j in range(HC // 16):
                msg_b[e, pl.ds(16 * j, 16)] = rs[j] * ps[j // 2]
            dv = jnp.where(lane == 0, ps[0], 0.0)
            dv = jnp.where(lane == 1, ps[1], dv)
            dv = jnp.where(lane == 2, ps[2], dv)
            dv = jnp.where(lane == 3, ps[3], dv)
            dsts = dst_v[pl.ds(16 * (e // 16), 16)]
            m8f = (dsts.at[jnp.full((16,), e % 16, jnp.int32)].get(
                mode="promise_in_bounds") & 7).astype(jnp.float32)
            for j in range(HC // 16):
                fac = jnp.maximum(1.0 - jnp.abs(m8f - float(j)), 0.0)
                den_b[e, pl.ds(16 * j, 16)] = dv * fac
            return cc2
        lax.fori_loop(0, CHUNK, _edge, 0)

        pltpu.sync_copy(msg_b, acc.at[dst_v], add=True)
        pltpu.sync_copy(den_b, acc.at[didx_v], add=True)
        return c
    lax.fori_loop(0, NCHUNK, _chunk, 0)

    plsc.subcore_barrier()
    def _wb(j, c):
        r = pl.multiple_of(row0 + j * CHUNK, CHUNK)
        pltpu.sync_copy(acc.at[pl.ds(r, CHUNK)], msg_b)
        pltpu.sync_copy(msg_b, num_out.at[cid, pl.ds(r, CHUNK)])
        return c
    lax.fori_loop(0, NRPT // CHUNK, _wb, 0)
    dr0 = sid * DRPT
    for j in (0, DRPT - CHUNK):
        pltpu.sync_copy(acc.at[pl.ds(NUM_ROWS + dr0 + j, CHUNK)], den_b)
        pltpu.sync_copy(den_b, den_out.at[cid, pl.ds(dr0 + j, CHUNK)])


def _sc_edges(xs, xd, src, dst, att_flat):
    mesh = plsc.VectorSubcoreMesh(core_axis_name="c", subcore_axis_name="s")
    return pl.kernel(
        _sc_edges_body,
        out_type=[
            jax.ShapeDtypeStruct((NCORE, NUM_ROWS, HC), jnp.float32),
            jax.ShapeDtypeStruct((NCORE, DEN_ROWS, HC), jnp.float32),
        ],
        mesh=mesh,
        scratch_types=[
            pltpu.VMEM((CHUNK,), jnp.int32),
            pltpu.VMEM((CHUNK,), jnp.int32),
            pltpu.VMEM((CHUNK,), jnp.int32),
            pltpu.VMEM((CHUNK, HC), jnp.float32),
            pltpu.VMEM((CHUNK, HC), jnp.float32),
            pltpu.VMEM((CHUNK, HC), jnp.float32),
            pltpu.VMEM((CHUNK, HC), jnp.float32),
            pltpu.VMEM((HC,), jnp.float32),
            pltpu.VMEM_SHARED((ACC_T, HC), jnp.float32),
            pltpu.SemaphoreType.DMA,
            pltpu.SemaphoreType.DMA,
        ],
    )(xs, xd, src, dst, att_flat)


def _combine_body(num_ref, den_ref, bias_ref, out_ref):
    num = num_ref[0] + num_ref[1]
    den = den_ref[0] + den_ref[1]
    row = lax.broadcasted_iota(jnp.int32, (16, HC), 0)
    col = lax.broadcasted_iota(jnp.int32, (16, HC), 1)
    sel = jnp.where(row == col // CC, 1.0, 0.0)
    den_b = jnp.dot(den, sel, preferred_element_type=jnp.float32)
    out_ref[...] = num / den_b + bias_ref[...]


def _combine(num, den16, bias2d):
    rows = 400
    grid = NN // rows
    return pl.pallas_call(
        _combine_body,
        grid=(grid,),
        in_specs=[
            pl.BlockSpec((NCORE, rows, HC), lambda i: (0, i, 0)),
            pl.BlockSpec((NCORE, rows, 16), lambda i: (0, i, 0)),
            pl.BlockSpec((1, HC), lambda i: (0, 0)),
        ],
        out_specs=pl.BlockSpec((rows, HC), lambda i: (i, 0)),
        out_shape=jax.ShapeDtypeStruct((NN, HC), jnp.float32),
    )(num, den16, bias2d)


@jax.jit
def kernel(x, edge_index, W_src, W_dst, att, bias):
    xs, xd = _project(x, W_src, W_dst)
    loops = jnp.arange(NN, dtype=jnp.int32)
    pad = EPAD - ETOT
    src = jnp.concatenate(
        [edge_index[0].astype(jnp.int32), loops,
         jnp.zeros((pad,), jnp.int32)])
    dst = jnp.concatenate(
        [edge_index[1].astype(jnp.int32), loops,
         jnp.full((pad,), NN, jnp.int32)])
    att_flat = att.reshape(HC)
    num, den = _sc_edges(xs, xd, src, dst, att_flat)
    den16 = den.reshape(NCORE, NUM_ROWS, 16)
    out = _combine(num, den16, bias.reshape(1, HC))
    return out


# speedup vs baseline: 48.9602x; 1.3468x over previous
import jax
import jax.numpy as jnp
from jax import lax
from jax.experimental import pallas as pl
from jax.experimental.pallas import tpu as pltpu
from jax.experimental.pallas import tpu_sc as plsc

NN = 10000
EE = 320000
DD = 128
HH = 4
CC = 32
HC = HH * CC
NEG = 0.2

NCORE = 2
NSUB = 16
NTILE = NCORE * NSUB

CHUNK = 32
ETOT = EE + NN
NCHUNK = -(-ETOT // (NTILE * CHUNK * 2)) * 2
EPT = NCHUNK * CHUNK
EPAD = NTILE * EPT
WB = 64

NUM_ROWS = 10240
DEN_ROWS = 1280
ACC_T = NUM_ROWS + DEN_ROWS
NRPT = NUM_ROWS // NSUB
DRPT = DEN_ROWS // NSUB


def _mm_body(x_ref, ws_ref, wd_ref, xs_ref, xd_ref):
    x = x_ref[...]
    xs_ref[...] = jnp.dot(x, ws_ref[...], preferred_element_type=jnp.float32)
    xd_ref[...] = jnp.dot(x, wd_ref[...], preferred_element_type=jnp.float32)


def _project(x, w_src, w_dst):
    rows = 1000
    grid = NN // rows
    return pl.pallas_call(
        _mm_body,
        grid=(grid,),
        in_specs=[
            pl.BlockSpec((rows, DD), lambda i: (i, 0)),
            pl.BlockSpec((DD, HC), lambda i: (0, 0)),
            pl.BlockSpec((DD, HC), lambda i: (0, 0)),
        ],
        out_specs=[
            pl.BlockSpec((rows, HC), lambda i: (i, 0)),
            pl.BlockSpec((rows, HC), lambda i: (i, 0)),
        ],
        out_shape=[jax.ShapeDtypeStruct((NN, HC), jnp.float32)] * 2,
    )(x, w_src, w_dst)


def _sc_edges_body(xs_hbm, xd_hbm, pk_hbm, att_hbm,
                   num_out, den_out,
                   pidx0, pidx1, cidx0, cidx1, xs0, xs1, xd0, xd1,
                   md0, md1, att_v,
                   acc, sgx0, sgx1, sgd0, sgd1, ssc0, ssc1):
    cid = lax.axis_index("c")
    sid = lax.axis_index("s")
    zero16 = jnp.zeros((16,), jnp.float32)
    pidx = (pidx0, pidx1)
    cidx = (cidx0, cidx1)
    xsb = (xs0, xs1)
    xdb = (xd0, xd1)
    mdb = (md0, md1)
    sgx = (sgx0, sgx1)
    sgd = (sgd0, sgd1)
    ssc = (ssc0, ssc1)

    def _zrow(r, c):
        for j in range(HC // 16):
            md0[r, pl.ds(16 * j, 16)] = zero16
        return c
    lax.fori_loop(0, WB, _zrow, 0)

    row0 = sid * NRPT
    for j in range(NRPT // WB):
        pltpu.sync_copy(md0, acc.at[pl.ds(row0 + j * WB, WB)])
    d0 = NUM_ROWS + sid * DRPT
    pltpu.sync_copy(md0, acc.at[pl.ds(d0, WB)])
    pltpu.sync_copy(md0, acc.at[pl.ds(d0 + DRPT - WB, WB)])
    plsc.subcore_barrier()

    pltpu.sync_copy(att_hbm, att_v)
    att_regs = [att_v[pl.ds(16 * j, 16)] for j in range(HC // 16)]
    lane = lax.broadcasted_iota(jnp.int32, (16,), 0)

    tile = cid * NSUB + sid
    base_w = tile * NCHUNK * 2 * CHUNK

    def _load_issue(n, p):
        off = pl.multiple_of(base_w + n * 2 * CHUNK, 2 * CHUNK)
        pltpu.sync_copy(pk_hbm.at[pl.ds(off, 2 * CHUNK)], pidx[p])
        pltpu.async_copy(xs_hbm.at[pidx[p].at[pl.ds(0, CHUNK)]], xsb[p],
                         sgx[p])
        pltpu.async_copy(xd_hbm.at[pidx[p].at[pl.ds(CHUNK, CHUNK)]], xdb[p],
                         sgd[p])

    def _wait_gathers(p):
        pltpu.make_async_copy(xs_hbm.at[pidx[p].at[pl.ds(0, CHUNK)]],
                              xsb[p], sgx[p]).wait()
        pltpu.make_async_copy(xd_hbm.at[pidx[p].at[pl.ds(CHUNK, CHUNK)]],
                              xdb[p], sgd[p]).wait()

    def _wait_scatter(p):
        pltpu.make_async_copy(mdb[p], acc.at[cidx[p]], ssc[p]).wait()

    def _compute(n, p):
        xs_b, xd_b, md_b, pidx_p, cidx_p = xsb[p], xdb[p], mdb[p], pidx[p], cidx[p]
        for j in range(CHUNK // 16):
            d = pidx_p[pl.ds(CHUNK + 16 * j, 16)]
            cidx_p[pl.ds(16 * j, 16)] = d
            cidx_p[pl.ds(CHUNK + 16 * j, 16)] = NUM_ROWS + (d >> 3)

        def _edge(e, cc2):
            rs = [xs_b[e, pl.ds(16 * j, 16)] for j in range(HC // 16)]
            rd = [xd_b[e, pl.ds(16 * j, 16)] for j in range(HC // 16)]
            ps = []
            for h in range(HH):
                acc_h = None
                for k in (2 * h, 2 * h + 1):
                    sv = rs[k] + rd[k]
                    lv = jnp.where(sv >= 0.0, sv, sv * NEG) * att_regs[k]
                    acc_h = lv if acc_h is None else acc_h + lv
                for sh in (1, 2, 4, 8):
                    acc_h = acc_h + acc_h.at[lane ^ sh].get(
                        mode="promise_in_bounds")
                ps.append(jnp.exp(acc_h))
            for j in range(HC // 16):
                md_b[e, pl.ds(16 * j, 16)] = rs[j] * ps[j // 2]
            dv = jnp.where(lane == 0, ps[0], 0.0)
            dv = jnp.where(lane == 1, ps[1], dv)
            dv = jnp.where(lane == 2, ps[2], dv)
            dv = jnp.where(lane == 3, ps[3], dv)
            dsts = pidx_p[pl.ds(CHUNK + 16 * (e // 16), 16)]
            m8f = (dsts.at[jnp.full((16,), e % 16, jnp.int32)].get(
                mode="promise_in_bounds") & 7).astype(jnp.float32)
            for j in range(HC // 16):
                fac = jnp.maximum(1.0 - jnp.abs(m8f - float(j)), 0.0)
                md_b[CHUNK + e, pl.ds(16 * j, 16)] = dv * fac
            return cc2
        lax.fori_loop(0, CHUNK, _edge, 0)

    _load_issue(0, 0)

    def _pair(k, c):
        for p in (0, 1):
            n = 2 * k + p

            @pl.when(n + 1 < NCHUNK)
            def _():
                _load_issue(n + 1, 1 - p)

            _wait_gathers(p)

            @pl.when(n >= 2)
            def _():
                _wait_scatter(p)

            _compute(n, p)
            pltpu.async_copy(mdb[p], acc.at[cidx[p]], ssc[p], add=True)
        return c
    lax.fori_loop(0, NCHUNK // 2, _pair, 0)
    _wait_scatter(0)
    _wait_scatter(1)

    plsc.subcore_barrier()
    def _wb(j, c):
        r = pl.multiple_of(row0 + j * WB, WB)
        pltpu.sync_copy(acc.at[pl.ds(r, WB)], md0)
        pltpu.sync_copy(md0, num_out.at[cid, pl.ds(r, WB)])
        return c
    lax.fori_loop(0, NRPT // WB, _wb, 0)
    dr0 = sid * DRPT
    for j in (0, DRPT - WB):
        pltpu.sync_copy(acc.at[pl.ds(NUM_ROWS + dr0 + j, WB)], md0)
        pltpu.sync_copy(md0, den_out.at[cid, pl.ds(dr0 + j, WB)])


def _sc_edges(xs, xd, pk, att_flat):
    mesh = plsc.VectorSubcoreMesh(core_axis_name="c", subcore_axis_name="s")
    return pl.kernel(
        _sc_edges_body,
        out_type=[
            jax.ShapeDtypeStruct((NCORE, NUM_ROWS, HC), jnp.float32),
            jax.ShapeDtypeStruct((NCORE, DEN_ROWS, HC), jnp.float32),
        ],
        mesh=mesh,
        scratch_types=[
            pltpu.VMEM((2 * CHUNK,), jnp.int32),
            pltpu.VMEM((2 * CHUNK,), jnp.int32),
            pltpu.VMEM((2 * CHUNK,), jnp.int32),
            pltpu.VMEM((2 * CHUNK,), jnp.int32),
            pltpu.VMEM((CHUNK, HC), jnp.float32),
            pltpu.VMEM((CHUNK, HC), jnp.float32),
            pltpu.VMEM((CHUNK, HC), jnp.float32),
            pltpu.VMEM((CHUNK, HC), jnp.float32),
            pltpu.VMEM((2 * CHUNK, HC), jnp.float32),
            pltpu.VMEM((2 * CHUNK, HC), jnp.float32),
            pltpu.VMEM((HC,), jnp.float32),
            pltpu.VMEM_SHARED((ACC_T, HC), jnp.float32),
            pltpu.SemaphoreType.DMA,
            pltpu.SemaphoreType.DMA,
            pltpu.SemaphoreType.DMA,
            pltpu.SemaphoreType.DMA,
            pltpu.SemaphoreType.DMA,
            pltpu.SemaphoreType.DMA,
        ],
    )(xs, xd, pk, att_flat)


def _combine_body(num_ref, den_ref, bias_ref, out_ref):
    num = num_ref[0] + num_ref[1]
    den = den_ref[0] + den_ref[1]
    row = lax.broadcasted_iota(jnp.int32, (16, HC), 0)
    col = lax.broadcasted_iota(jnp.int32, (16, HC), 1)
    sel = jnp.where(row == col // CC, 1.0, 0.0)
    den_b = jnp.dot(den, sel, preferred_element_type=jnp.float32)
    out_ref[...] = num / den_b + bias_ref[...]


def _combine(num, den16, bias2d):
    rows = 400
    grid = NN // rows
    return pl.pallas_call(
        _combine_body,
        grid=(grid,),
        in_specs=[
            pl.BlockSpec((NCORE, rows, HC), lambda i: (0, i, 0)),
            pl.BlockSpec((NCORE, rows, 16), lambda i: (0, i, 0)),
            pl.BlockSpec((1, HC), lambda i: (0, 0)),
        ],
        out_specs=pl.BlockSpec((rows, HC), lambda i: (i, 0)),
        out_shape=jax.ShapeDtypeStruct((NN, HC), jnp.float32),
    )(num, den16, bias2d)


@jax.jit
def kernel(x, edge_index, W_src, W_dst, att, bias):
    xs, xd = _project(x, W_src, W_dst)
    loops = jnp.arange(NN, dtype=jnp.int32)
    pad = EPAD - ETOT
    src = jnp.concatenate(
        [edge_index[0].astype(jnp.int32), loops,
         jnp.zeros((pad,), jnp.int32)])
    dst = jnp.concatenate(
        [edge_index[1].astype(jnp.int32), loops,
         jnp.full((pad,), NN, jnp.int32)])
    pk = jnp.stack([src.reshape(-1, CHUNK), dst.reshape(-1, CHUNK)],
                   axis=1).reshape(-1)
    att_flat = att.reshape(HC)
    num, den = _sc_edges(xs, xd, pk, att_flat)
    den16 = den.reshape(NCORE, NUM_ROWS, 16)
    out = _combine(num, den16, bias.reshape(1, HC))
    return out


# speedup vs baseline: 64.1466x; 1.3102x over previous
import jax
import jax.numpy as jnp
from jax import lax
from jax.experimental import pallas as pl
from jax.experimental.pallas import tpu as pltpu
from jax.experimental.pallas import tpu_sc as plsc

NN = 10000
EE = 320000
DD = 128
HH = 4
CC = 32
HC = HH * CC
NEG = 0.2

NCORE = 2
NSUB = 16
NTILE = NCORE * NSUB

CHUNK = 32
ETOT = EE + NN
NCHUNK = -(-ETOT // (NTILE * CHUNK * 2)) * 2
EPT = NCHUNK * CHUNK
EPAD = NTILE * EPT
WB = 64

NUM_ROWS = 10240
DEN_ROWS = 1280
ACC_T = NUM_ROWS + DEN_ROWS
NRPT = NUM_ROWS // NSUB
DRPT = DEN_ROWS // NSUB


def _mm_body(x_ref, ws_ref, wd_ref, xs_ref, xd_ref):
    x = x_ref[...]
    xs_ref[...] = jnp.dot(x, ws_ref[...], preferred_element_type=jnp.float32)
    xd_ref[...] = jnp.dot(x, wd_ref[...], preferred_element_type=jnp.float32)


def _project(x, w_src, w_dst):
    rows = 1000
    grid = NN // rows
    return pl.pallas_call(
        _mm_body,
        grid=(grid,),
        in_specs=[
            pl.BlockSpec((rows, DD), lambda i: (i, 0)),
            pl.BlockSpec((DD, HC), lambda i: (0, 0)),
            pl.BlockSpec((DD, HC), lambda i: (0, 0)),
        ],
        out_specs=[
            pl.BlockSpec((rows, HC), lambda i: (i, 0)),
            pl.BlockSpec((rows, HC), lambda i: (i, 0)),
        ],
        out_shape=[jax.ShapeDtypeStruct((NN, HC), jnp.float32)] * 2,
    )(x, w_src, w_dst)


def _sc_edges_body(xs_hbm, xd_hbm, pk_hbm, att_hbm,
                   num_out, den_out,
                   pidx0, pidx1, cidx0, cidx1, xs0, xs1, xd0, xd1,
                   md0, md1, att_v,
                   acc, sgx0, sgx1, sgd0, sgd1, ssc0, ssc1):
    cid = lax.axis_index("c")
    sid = lax.axis_index("s")
    zero16 = jnp.zeros((16,), jnp.float32)
    pidx = (pidx0, pidx1)
    cidx = (cidx0, cidx1)
    xsb = (xs0, xs1)
    xdb = (xd0, xd1)
    mdb = (md0, md1)
    sgx = (sgx0, sgx1)
    sgd = (sgd0, sgd1)
    ssc = (ssc0, ssc1)

    def _zrow(r, c):
        for j in range(HC // 16):
            md0[r, pl.ds(16 * j, 16)] = zero16
        return c
    lax.fori_loop(0, WB, _zrow, 0)

    row0 = sid * NRPT
    for j in range(NRPT // WB):
        pltpu.sync_copy(md0, acc.at[pl.ds(row0 + j * WB, WB)])
    d0 = NUM_ROWS + sid * DRPT
    pltpu.sync_copy(md0, acc.at[pl.ds(d0, WB)])
    pltpu.sync_copy(md0, acc.at[pl.ds(d0 + DRPT - WB, WB)])
    plsc.subcore_barrier()

    pltpu.sync_copy(att_hbm, att_v)
    att_regs = [att_v[pl.ds(16 * j, 16)] for j in range(HC // 16)]
    lane = lax.broadcasted_iota(jnp.int32, (16,), 0)

    tile = cid * NSUB + sid
    base_w = tile * NCHUNK * 2 * CHUNK

    def _load_issue(n, p):
        off = pl.multiple_of(base_w + n * 2 * CHUNK, 2 * CHUNK)
        pltpu.sync_copy(pk_hbm.at[pl.ds(off, 2 * CHUNK)], pidx[p])
        pltpu.async_copy(xs_hbm.at[pidx[p].at[pl.ds(0, CHUNK)]], xsb[p],
                         sgx[p])
        pltpu.async_copy(xd_hbm.at[pidx[p].at[pl.ds(CHUNK, CHUNK)]], xdb[p],
                         sgd[p])

    def _wait_gathers(p):
        pltpu.make_async_copy(xs_hbm.at[pidx[p].at[pl.ds(0, CHUNK)]],
                              xsb[p], sgx[p]).wait()
        pltpu.make_async_copy(xd_hbm.at[pidx[p].at[pl.ds(CHUNK, CHUNK)]],
                              xdb[p], sgd[p]).wait()

    def _wait_scatter(p):
        pltpu.make_async_copy(mdb[p], acc.at[cidx[p]], ssc[p]).wait()

    def _compute(n, p):
        xs_b, xd_b, md_b, pidx_p, cidx_p = xsb[p], xdb[p], mdb[p], pidx[p], cidx[p]
        for j in range(CHUNK // 16):
            d = pidx_p[pl.ds(CHUNK + 16 * j, 16)]
            cidx_p[pl.ds(16 * j, 16)] = d
            cidx_p[pl.ds(CHUNK + 16 * j, 16)] = NUM_ROWS + (d >> 3)

        @plsc.parallel_loop(0, CHUNK, 1, unroll=2)
        def _edge(e):
            rs = [xs_b[e, pl.ds(16 * j, 16)] for j in range(HC // 16)]
            rd = [xd_b[e, pl.ds(16 * j, 16)] for j in range(HC // 16)]
            ps = []
            for h in range(HH):
                acc_h = None
                for k in (2 * h, 2 * h + 1):
                    sv = rs[k] + rd[k]
                    lv = jnp.where(sv >= 0.0, sv, sv * NEG) * att_regs[k]
                    acc_h = lv if acc_h is None else acc_h + lv
                for sh in (1, 2, 4, 8):
                    acc_h = acc_h + acc_h.at[lane ^ sh].get(
                        mode="promise_in_bounds")
                ps.append(jnp.exp(acc_h))
            for j in range(HC // 16):
                md_b[e, pl.ds(16 * j, 16)] = rs[j] * ps[j // 2]
            dv = jnp.where(lane == 0, ps[0], 0.0)
            dv = jnp.where(lane == 1, ps[1], dv)
            dv = jnp.where(lane == 2, ps[2], dv)
            dv = jnp.where(lane == 3, ps[3], dv)
            dsts = pidx_p[pl.ds(CHUNK + 16 * (e // 16), 16)]
            m8f = (dsts.at[jnp.full((16,), e % 16, jnp.int32)].get(
                mode="promise_in_bounds") & 7).astype(jnp.float32)
            for j in range(HC // 16):
                fac = jnp.maximum(1.0 - jnp.abs(m8f - float(j)), 0.0)
                md_b[CHUNK + e, pl.ds(16 * j, 16)] = dv * fac

    _load_issue(0, 0)

    def _pair(k, c):
        for p in (0, 1):
            n = 2 * k + p

            @pl.when(n + 1 < NCHUNK)
            def _():
                _load_issue(n + 1, 1 - p)

            _wait_gathers(p)

            @pl.when(n >= 2)
            def _():
                _wait_scatter(p)

            _compute(n, p)
            pltpu.async_copy(mdb[p], acc.at[cidx[p]], ssc[p], add=True)
        return c
    lax.fori_loop(0, NCHUNK // 2, _pair, 0)
    _wait_scatter(0)
    _wait_scatter(1)

    plsc.subcore_barrier()
    def _wb(j, c):
        r = pl.multiple_of(row0 + j * WB, WB)
        pltpu.sync_copy(acc.at[pl.ds(r, WB)], md0)
        pltpu.sync_copy(md0, num_out.at[cid, pl.ds(r, WB)])
        return c
    lax.fori_loop(0, NRPT // WB, _wb, 0)
    dr0 = sid * DRPT
    for j in (0, DRPT - WB):
        pltpu.sync_copy(acc.at[pl.ds(NUM_ROWS + dr0 + j, WB)], md0)
        pltpu.sync_copy(md0, den_out.at[cid, pl.ds(dr0 + j, WB)])


def _sc_edges(xs, xd, pk, att_flat):
    mesh = plsc.VectorSubcoreMesh(core_axis_name="c", subcore_axis_name="s")
    return pl.kernel(
        _sc_edges_body,
        out_type=[
            jax.ShapeDtypeStruct((NCORE, NUM_ROWS, HC), jnp.float32),
            jax.ShapeDtypeStruct((NCORE, DEN_ROWS, HC), jnp.float32),
        ],
        mesh=mesh,
        scratch_types=[
            pltpu.VMEM((2 * CHUNK,), jnp.int32),
            pltpu.VMEM((2 * CHUNK,), jnp.int32),
            pltpu.VMEM((2 * CHUNK,), jnp.int32),
            pltpu.VMEM((2 * CHUNK,), jnp.int32),
            pltpu.VMEM((CHUNK, HC), jnp.float32),
            pltpu.VMEM((CHUNK, HC), jnp.float32),
            pltpu.VMEM((CHUNK, HC), jnp.float32),
            pltpu.VMEM((CHUNK, HC), jnp.float32),
            pltpu.VMEM((2 * CHUNK, HC), jnp.float32),
            pltpu.VMEM((2 * CHUNK, HC), jnp.float32),
            pltpu.VMEM((HC,), jnp.float32),
            pltpu.VMEM_SHARED((ACC_T, HC), jnp.float32),
            pltpu.SemaphoreType.DMA,
            pltpu.SemaphoreType.DMA,
            pltpu.SemaphoreType.DMA,
            pltpu.SemaphoreType.DMA,
            pltpu.SemaphoreType.DMA,
            pltpu.SemaphoreType.DMA,
        ],
    )(xs, xd, pk, att_flat)


def _combine_body(num_ref, den_ref, bias_ref, out_ref):
    num = num_ref[0] + num_ref[1]
    den = den_ref[0] + den_ref[1]
    row = lax.broadcasted_iota(jnp.int32, (16, HC), 0)
    col = lax.broadcasted_iota(jnp.int32, (16, HC), 1)
    sel = jnp.where(row == col // CC, 1.0, 0.0)
    den_b = jnp.dot(den, sel, preferred_element_type=jnp.float32)
    out_ref[...] = num / den_b + bias_ref[...]


def _combine(num, den16, bias2d):
    rows = 400
    grid = NN // rows
    return pl.pallas_call(
        _combine_body,
        grid=(grid,),
        in_specs=[
            pl.BlockSpec((NCORE, rows, HC), lambda i: (0, i, 0)),
            pl.BlockSpec((NCORE, rows, 16), lambda i: (0, i, 0)),
            pl.BlockSpec((1, HC), lambda i: (0, 0)),
        ],
        out_specs=pl.BlockSpec((rows, HC), lambda i: (i, 0)),
        out_shape=jax.ShapeDtypeStruct((NN, HC), jnp.float32),
    )(num, den16, bias2d)


@jax.jit
def kernel(x, edge_index, W_src, W_dst, att, bias):
    xs, xd = _project(x, W_src, W_dst)
    loops = jnp.arange(NN, dtype=jnp.int32)
    pad = EPAD - ETOT
    src = jnp.concatenate(
        [edge_index[0].astype(jnp.int32), loops,
         jnp.zeros((pad,), jnp.int32)])
    dst = jnp.concatenate(
        [edge_index[1].astype(jnp.int32), loops,
         jnp.full((pad,), NN, jnp.int32)])
    pk = jnp.stack([src.reshape(-1, CHUNK), dst.reshape(-1, CHUNK)],
                   axis=1).reshape(-1)
    att_flat = att.reshape(HC)
    num, den = _sc_edges(xs, xd, pk, att_flat)
    den16 = den.reshape(NCORE, NUM_ROWS, 16)
    out = _combine(num, den16, bias.reshape(1, HC))
    return out
